# Initial kernel scaffold; baseline (speedup 1.0000x reference)
import jax
import jax.numpy as jnp
from jax.experimental import pallas as pl


def kernel(var_lp_f, con_lp_f, edge_lp_f_wo_ss, var_learned_f, con_learned_f, edge_learned_f, params, edge_index_var_con, batch_index_var, batch_index_con, batch_index_edge):
    raise NotImplementedError("write your pallas kernel here")


# speedup vs baseline: 6.7332x; 6.7332x over previous
import functools

import jax
import jax.numpy as jnp
from jax import lax
from jax.experimental import pallas as pl
from jax.experimental.pallas import tpu as pltpu
from jax.experimental.pallas import tpu_sc as plsc

NV = 100000
NC = 50000
E = 1600000
DV = 16
NLP = 3

WDC = 8
CHUNK = 1000
BR_N = 2000
BR_E = 4000

F32 = jnp.float32


def _norm_body(x_ref, o_ref):
    x = x_ref[...]
    r = 1.0 / jnp.maximum(jnp.sqrt(jnp.sum(x * x)), 1e-6)
    o_ref[...] = jnp.broadcast_to(r, (1, 1))


def _rnorm(col):
    return pl.pallas_call(
        _norm_body,
        out_shape=jax.ShapeDtypeStruct((1, 1), F32),
        in_specs=[pl.BlockSpec((800, 125), lambda: (0, 0))],
        out_specs=pl.BlockSpec((1, 1), lambda: (0, 0)),
    )(col)


def _proj_body(splits, xl_ref, xp_ref, wa_ref, wb_ref, b_ref, *o_refs):
    y = (jnp.dot(xl_ref[...], wa_ref[...], preferred_element_type=F32)
         + jnp.dot(xp_ref[...], wb_ref[...], preferred_element_type=F32)
         + b_ref[...])
    off = 0
    for o_ref, w in zip(o_refs, splits):
        o_ref[...] = y[:, off:off + w]
        off += w


def _proj(xl, xp, W, b, splits, br):
    n, kl = xl.shape
    kp = xp.shape[1]
    f = W.shape[1]
    wa, wb = W[:kl], W[kl:]
    return pl.pallas_call(
        functools.partial(_proj_body, splits),
        out_shape=[jax.ShapeDtypeStruct((n, w), F32) for w in splits],
        grid=(n // br,),
        in_specs=[
            pl.BlockSpec((br, kl), lambda i: (i, 0)),
            pl.BlockSpec((br, kp), lambda i: (i, 0)),
            pl.BlockSpec((kl, f), lambda i: (0, 0)),
            pl.BlockSpec((kp, f), lambda i: (0, 0)),
            pl.BlockSpec((1, f), lambda i: (0, 0)),
        ],
        out_specs=[pl.BlockSpec((br, w), lambda i: (i, 0)) for w in splits],
    )(xl, xp, wa, wb, b.reshape(1, f))


def _att_body(qg_ref, kvg_ref, ec_ref, o_num_ref, o_dc_ref):
    q = qg_ref[...]
    kv = kvg_ref[...]
    ec = ec_ref[...]
    kj = kv[:, :DV] + ec
    vj = kv[:, DV:] + ec
    ex = jnp.exp(jnp.sum(q * kj, axis=1, keepdims=True) * 0.25)
    br = q.shape[0]
    o_num_ref[...] = vj * ex
    o_dc_ref[...] = jnp.concatenate(
        [ex, jnp.ones((br, 1), F32), jnp.zeros((br, WDC - 2), F32)], axis=1)


def _attention_payload(qg, kvg, ec):
    return pl.pallas_call(
        _att_body,
        out_shape=[jax.ShapeDtypeStruct((E, DV), F32),
                   jax.ShapeDtypeStruct((E, WDC), F32)],
        grid=(E // BR_E,),
        in_specs=[
            pl.BlockSpec((BR_E, DV), lambda i: (i, 0)),
            pl.BlockSpec((BR_E, 2 * DV), lambda i: (i, 0)),
            pl.BlockSpec((BR_E, DV), lambda i: (i, 0)),
        ],
        out_specs=[pl.BlockSpec((BR_E, DV), lambda i: (i, 0)),
                   pl.BlockSpec((BR_E, WDC), lambda i: (i, 0))],
    )(qg, kvg, ec)


def _finalize_body(nf, p0_ref, p1_ref, d0_ref, d1_ref, skip_ref, lp_ref,
                   wkva_ref, wkvb_ref, bkv_ref, w1a_ref, w1b_ref, b1_ref,
                   w2_ref, b2_ref, wfold_ref, o_node_ref, o_kv_ref, o_fold_ref):
    num = p0_ref[...] + p1_ref[...]
    dc = d0_ref[...] + d1_ref[...]
    den = dc[:, 0:1]
    cnt = dc[:, 1:2]
    node = jax.nn.relu(num / (den + 1e-16) / jnp.maximum(cnt, 1.0) + skip_ref[...])
    o_node_ref[...] = node
    lp = lp_ref[...]
    o_kv_ref[...] = (jnp.dot(node, wkva_ref[...], preferred_element_type=F32)
                     + jnp.dot(lp, wkvb_ref[...], preferred_element_type=F32)
                     + bkv_ref[...])
    h = jax.nn.relu(jnp.dot(node, w1a_ref[...], preferred_element_type=F32)
                    + jnp.dot(lp, w1b_ref[...], preferred_element_type=F32)
                    + b1_ref[...])
    h2 = jax.nn.relu(jnp.dot(h, w2_ref[...], preferred_element_type=F32) + b2_ref[...])
    o_fold_ref[...] = jnp.dot(h2, wfold_ref[...], preferred_element_type=F32)


def _finalize(parts_num, parts_dc, skip, lp, wkv, bkv, w1, b1, w2, b2, wfold,
              n, kv_width):
    nb = n // BR_N
    return pl.pallas_call(
        functools.partial(_finalize_body, n),
        out_shape=[jax.ShapeDtypeStruct((n, DV), F32),
                   jax.ShapeDtypeStruct((n, kv_width), F32),
                   jax.ShapeDtypeStruct((n, DV), F32)],
        grid=(nb,),
        in_specs=[
            pl.BlockSpec((BR_N, DV), lambda i: (i, 0)),
            pl.BlockSpec((BR_N, DV), lambda i, nb=nb: (i + nb, 0)),
            pl.BlockSpec((BR_N, WDC), lambda i: (i, 0)),
            pl.BlockSpec((BR_N, WDC), lambda i, nb=nb: (i + nb, 0)),
            pl.BlockSpec((BR_N, DV), lambda i: (i, 0)),
            pl.BlockSpec((BR_N, NLP), lambda i: (i, 0)),
            pl.BlockSpec((DV, kv_width), lambda i: (0, 0)),
            pl.BlockSpec((NLP, kv_width), lambda i: (0, 0)),
            pl.BlockSpec((1, kv_width), lambda i: (0, 0)),
            pl.BlockSpec((DV, DV), lambda i: (0, 0)),
            pl.BlockSpec((NLP, DV), lambda i: (0, 0)),
            pl.BlockSpec((1, DV), lambda i: (0, 0)),
            pl.BlockSpec((DV, DV), lambda i: (0, 0)),
            pl.BlockSpec((1, DV), lambda i: (0, 0)),
            pl.BlockSpec((DV, DV), lambda i: (0, 0)),
        ],
        out_specs=[pl.BlockSpec((BR_N, DV), lambda i: (i, 0)),
                   pl.BlockSpec((BR_N, kv_width), lambda i: (i, 0)),
                   pl.BlockSpec((BR_N, DV), lambda i: (i, 0))],
    )(parts_num, parts_num, parts_dc, parts_dc, skip, lp,
      wkv[:DV], wkv[DV:], bkv.reshape(1, -1),
      w1[:DV], w1[DV:], b1.reshape(1, -1), w2, b2.reshape(1, -1), wfold)


def _edge_out_body(g1_ref, vcg_ref, ccg_ref, w2_ref, b2_ref, o_ref):
    h = jax.nn.relu(g1_ref[...] + vcg_ref[...] + ccg_ref[...])
    o_ref[...] = jax.nn.relu(
        jnp.dot(h, w2_ref[...], preferred_element_type=F32) + b2_ref[...])


def _edge_out(g1, vcg, ccg, w2, b2):
    return pl.pallas_call(
        _edge_out_body,
        out_shape=jax.ShapeDtypeStruct((E, DV), F32),
        grid=(E // BR_E,),
        in_specs=[pl.BlockSpec((BR_E, DV), lambda i: (i, 0))] * 3
        + [pl.BlockSpec((DV, DV), lambda i: (0, 0)),
           pl.BlockSpec((1, DV), lambda i: (0, 0))],
        out_specs=pl.BlockSpec((BR_E, DV), lambda i: (i, 0)),
    )(g1, vcg, ccg, w2, b2.reshape(1, DV))


def _sc_mesh():
    info = plsc.get_sparse_core_info()
    return (plsc.VectorSubcoreMesh(core_axis_name="c", subcore_axis_name="s"),
            info.num_cores, info.num_subcores)


def _gather2(tab1, idx1, tab2, idx2):
    mesh, ncores, nsub = _sc_mesh()
    nw = ncores * nsub
    ne_t = E // nw
    nch = ne_t // CHUNK
    w1 = tab1.shape[1]
    w2 = tab2.shape[1]

    def body(tab1_hbm, idx1_hbm, tab2_hbm, idx2_hbm, o1_hbm, o2_hbm,
             i1, i2, r1, r2, s1, s2):
        wid = lax.axis_index("s") * ncores + lax.axis_index("c")

        def step(g, _):
            base = wid * ne_t + g * CHUNK
            pltpu.sync_copy(idx1_hbm.at[pl.ds(base, CHUNK)], i1)
            pltpu.sync_copy(idx2_hbm.at[pl.ds(base, CHUNK)], i2)
            c1 = pltpu.async_copy(tab1_hbm.at[i1], r1, s1)
            c2 = pltpu.async_copy(tab2_hbm.at[i2], r2, s2)
            c1.wait()
            c2.wait()
            pltpu.sync_copy(r1, o1_hbm.at[pl.ds(base, CHUNK)])
            pltpu.sync_copy(r2, o2_hbm.at[pl.ds(base, CHUNK)])
            return 0

        lax.fori_loop(0, nch, step, 0)

    return pl.kernel(
        body,
        out_type=[jax.ShapeDtypeStruct((E, w1), F32),
                  jax.ShapeDtypeStruct((E, w2), F32)],
        mesh=mesh,
        compiler_params=pltpu.CompilerParams(use_tc_tiling_on_sc=False),
        scratch_types=[
            pltpu.VMEM((CHUNK,), jnp.int32),
            pltpu.VMEM((CHUNK,), jnp.int32),
            pltpu.VMEM((CHUNK, w1), F32),
            pltpu.VMEM((CHUNK, w2), F32),
            pltpu.SemaphoreType.DMA,
            pltpu.SemaphoreType.DMA,
        ],
    )(tab1, idx1, tab2, idx2)


def _scatter_add(pays, idx, zeros_list, n):
    mesh, ncores, nsub = _sc_mesh()
    nw = ncores * nsub
    ne_t = E // nw
    nch = ne_t // CHUNK
    rpt = n // nsub
    widths = [p.shape[1] for p in pays]
    np_ = len(pays)

    def body(*refs):
        pay_hbms = refs[:np_]
        idx_hbm = refs[np_]
        zero_hbms = refs[np_ + 1:2 * np_ + 1]
        out_hbms = refs[2 * np_ + 1:3 * np_ + 1]
        idxv = refs[3 * np_ + 1]
        payvs = refs[3 * np_ + 2:4 * np_ + 2]
        accs = refs[4 * np_ + 2:]
        cid = lax.axis_index("c")
        sid = lax.axis_index("s")
        wid = sid * ncores + cid
        for z, a in zip(zero_hbms, accs):
            pltpu.sync_copy(z.at[pl.ds(sid * rpt, rpt)],
                            a.at[pl.ds(sid * rpt, rpt)])
        plsc.subcore_barrier()

        def step(g, _):
            base = wid * ne_t + g * CHUNK
            pltpu.sync_copy(idx_hbm.at[pl.ds(base, CHUNK)], idxv.at[0])
            for ph, pv, a in zip(pay_hbms, payvs, accs):
                pltpu.sync_copy(ph.at[pl.ds(base, CHUNK)], pv)
                pltpu.sync_copy(pv, a.at[idxv.at[0]], add=True)
            return 0

        lax.fori_loop(0, nch, step, 0)
        plsc.subcore_barrier()
        for a, o in zip(accs, out_hbms):
            pltpu.sync_copy(a.at[pl.ds(sid * rpt, rpt)],
                            o.at[pl.ds(cid * n + sid * rpt, rpt)])

    return pl.kernel(
        body,
        out_type=[jax.ShapeDtypeStruct((2 * n, w), F32) for w in widths],
        mesh=mesh,
        compiler_params=pltpu.CompilerParams(use_tc_tiling_on_sc=False),
        scratch_types=[pltpu.VMEM((1, CHUNK), jnp.int32)]
        + [pltpu.VMEM((CHUNK, w), F32) for w in widths]
        + [pltpu.VMEM_SHARED((n, w), F32) for w in widths],
    )(*pays, idx, *zeros_list)


def kernel(var_lp_f, con_lp_f, edge_lp_f_wo_ss, var_learned_f, con_learned_f,
           edge_learned_f, params, edge_index_var_con, batch_index_var,
           batch_index_con, batch_index_edge):
    p = params[0]
    src = edge_index_var_con[0]
    dst = edge_index_var_con[1]

    rnorm = _rnorm(var_lp_f[:, 2].reshape(800, 125))
    s = rnorm[0, 0]

    def s18(W):
        return W.at[18].set(W[18] * s)

    W64 = jnp.concatenate([s18(p["con"]["k"]["W"]), s18(p["con"]["v"]["W"]),
                           s18(p["var"]["q"]["W"]), s18(p["var"]["skip"]["W"])], axis=1)
    b64 = jnp.concatenate([p["con"]["k"]["b"], p["con"]["v"]["b"],
                           p["var"]["q"]["b"], p["var"]["skip"]["b"]])
    W32 = jnp.concatenate([s18(p["con"]["q"]["W"]), s18(p["con"]["skip"]["W"])], axis=1)
    b32 = jnp.concatenate([p["con"]["q"]["b"], p["con"]["skip"]["b"]])
    W1 = p["eu_e1"]["W"]
    W1a, W1b, W1c = W1[:19], W1[19:35], W1[35:51]
    W48 = jnp.concatenate([s18(p["con"]["e"]["W"]), s18(p["var"]["e"]["W"]),
                           s18(W1a)], axis=1)
    b48 = jnp.concatenate([jnp.zeros((32,), F32), p["eu_e1"]["b"]])
    Wkv_c = jnp.concatenate([s18(p["var"]["k"]["W"]), s18(p["var"]["v"]["W"])], axis=1)
    bkv_c = jnp.concatenate([p["var"]["k"]["b"], p["var"]["v"]["b"]])

    kv_v, q_v, skip_v = _proj(var_learned_f, var_lp_f, W64, b64,
                              (2 * DV, DV, DV), BR_N)
    q_c, skip_c = _proj(con_learned_f, con_lp_f, W32, b32, (DV, DV), BR_N)
    e_con, e_var, g1 = _proj(edge_learned_f, edge_lp_f_wo_ss, W48, b48,
                             (DV, DV, DV), BR_E)

    zc_num = jnp.zeros((NC, DV), F32)
    zc_dc = jnp.zeros((NC, WDC), F32)
    zv_num = jnp.zeros((NV, DV), F32)
    zv_dc = jnp.zeros((NV, WDC), F32)

    kvg, qg = _gather2(kv_v, src, q_c, dst)
    pay1n, pay1d = _attention_payload(qg, kvg, e_con)
    pc_num, pc_dc = _scatter_add([pay1n, pay1d], dst, [zc_num, zc_dc], NC)
    con_l, kv_c, cc2 = _finalize(pc_num, pc_dc, skip_c, con_lp_f, Wkv_c, bkv_c,
                                 s18(p["eu_c1"]["W"]), p["eu_c1"]["b"],
                                 p["eu_c2"]["W"], p["eu_c2"]["b"], W1c,
                                 NC, 2 * DV)

    kvg2, qg2 = _gather2(kv_c, dst, q_v, src)
    pay2n, pay2d = _attention_payload(qg2, kvg2, e_var)
    (pv_num,) = _scatter_add([pay2n], src, [zv_num], NV)
    (pv_dc,) = _scatter_add([pay2d], src, [zv_dc], NV)
    var_l, _unused, vc2 = _finalize(pv_num, pv_dc, skip_v, var_lp_f,
                                    jnp.zeros((19, DV), F32), jnp.zeros((DV,), F32),
                                    s18(p["eu_v1"]["W"]), p["eu_v1"]["b"],
                                    p["eu_v2"]["W"], p["eu_v2"]["b"], W1b,
                                    NV, DV)

    vcg, ccg = _gather2(vc2, src, cc2, dst)
    edge_l = _edge_out(g1, vcg, ccg, p["eu_e2"]["W"], p["eu_e2"]["b"])

    return (var_l, con_l, edge_l)


# speedup vs baseline: 6.7381x; 1.0007x over previous
import functools

import jax
import jax.numpy as jnp
from jax import lax
from jax.experimental import pallas as pl
from jax.experimental.pallas import tpu as pltpu
from jax.experimental.pallas import tpu_sc as plsc

NV = 100000
NC = 50000
E = 1600000
DV = 16
NLP = 3

WDC = 8
CHUNK = 400
HALF = NV // 2
VACC = HALF + 2000
BR_N = 2000
BRP = 2000

F32 = jnp.float32


def _norm_body(x_ref, o_ref):
    x = x_ref[...]
    r = 1.0 / jnp.maximum(jnp.sqrt(jnp.sum(x * x)), 1e-6)
    o_ref[...] = jnp.broadcast_to(r, (1, 1))


def _rnorm(col):
    return pl.pallas_call(
        _norm_body,
        out_shape=jax.ShapeDtypeStruct((1, 1), F32),
        in_specs=[pl.BlockSpec((800, 125), lambda: (0, 0))],
        out_specs=pl.BlockSpec((1, 1), lambda: (0, 0)),
    )(col)


def _proj_body(splits, xl_ref, xp_ref, wa_ref, wb_ref, b_ref, *o_refs):
    y = (jnp.dot(xl_ref[...], wa_ref[...], preferred_element_type=F32)
         + jnp.dot(xp_ref[...], wb_ref[...], preferred_element_type=F32)
         + b_ref[...])
    off = 0
    for o_ref, w in zip(o_refs, splits):
        o_ref[...] = y[:, off:off + w]
        off += w


def _proj(xl, xp, W, b, splits, br):
    n, kl = xl.shape
    kp = xp.shape[1]
    f = W.shape[1]
    wa, wb = W[:kl], W[kl:]
    return pl.pallas_call(
        functools.partial(_proj_body, splits),
        out_shape=[jax.ShapeDtypeStruct((n, w), F32) for w in splits],
        grid=(n // br,),
        in_specs=[
            pl.BlockSpec((br, kl), lambda i: (i, 0)),
            pl.BlockSpec((br, kp), lambda i: (i, 0)),
            pl.BlockSpec((kl, f), lambda i: (0, 0)),
            pl.BlockSpec((kp, f), lambda i: (0, 0)),
            pl.BlockSpec((1, f), lambda i: (0, 0)),
        ],
        out_specs=[pl.BlockSpec((br, w), lambda i: (i, 0)) for w in splits],
    )(xl, xp, wa, wb, b.reshape(1, f))


def _finalize_con_body(p0_ref, p1_ref, d0_ref, d1_ref, skip_ref, lp_ref,
                       wkva_ref, wkvb_ref, bkv_ref, w1a_ref, w1b_ref, b1_ref,
                       w2_ref, b2_ref, wfold_ref, o_node_ref, o_kv_ref,
                       o_fold_ref):
    num = p0_ref[...] + p1_ref[...]
    dc = d0_ref[...] + d1_ref[...]
    _finalize_common(num, dc, skip_ref, lp_ref, wkva_ref, wkvb_ref, bkv_ref,
                     w1a_ref, w1b_ref, b1_ref, w2_ref, b2_ref, wfold_ref,
                     o_node_ref, o_kv_ref, o_fold_ref)


def _finalize_var_body(p0_ref, d0_ref, skip_ref, lp_ref,
                       w1a_ref, w1b_ref, b1_ref, w2_ref, b2_ref, wfold_ref,
                       o_node_ref, o_fold_ref):
    num = p0_ref[...]
    dc = d0_ref[...]
    den = dc[:, 0:1]
    cnt = dc[:, 1:2]
    node = jax.nn.relu(num / (den + 1e-16) / jnp.maximum(cnt, 1.0)
                       + skip_ref[...])
    o_node_ref[...] = node
    lp = lp_ref[...]
    h = jax.nn.relu(jnp.dot(node, w1a_ref[...], preferred_element_type=F32)
                    + jnp.dot(lp, w1b_ref[...], preferred_element_type=F32)
                    + b1_ref[...])
    h2 = jax.nn.relu(jnp.dot(h, w2_ref[...], preferred_element_type=F32)
                     + b2_ref[...])
    o_fold_ref[...] = jnp.dot(h2, wfold_ref[...], preferred_element_type=F32)


def _finalize_common(num, dc, skip_ref, lp_ref, wkva_ref, wkvb_ref, bkv_ref,
                     w1a_ref, w1b_ref, b1_ref, w2_ref, b2_ref, wfold_ref,
                     o_node_ref, o_kv_ref, o_fold_ref):
    den = dc[:, 0:1]
    cnt = dc[:, 1:2]
    node = jax.nn.relu(num / (den + 1e-16) / jnp.maximum(cnt, 1.0)
                       + skip_ref[...])
    o_node_ref[...] = node
    lp = lp_ref[...]
    o_kv_ref[...] = (jnp.dot(node, wkva_ref[...], preferred_element_type=F32)
                     + jnp.dot(lp, wkvb_ref[...], preferred_element_type=F32)
                     + bkv_ref[...])
    h = jax.nn.relu(jnp.dot(node, w1a_ref[...], preferred_element_type=F32)
                    + jnp.dot(lp, w1b_ref[...], preferred_element_type=F32)
                    + b1_ref[...])
    h2 = jax.nn.relu(jnp.dot(h, w2_ref[...], preferred_element_type=F32)
                     + b2_ref[...])
    o_fold_ref[...] = jnp.dot(h2, wfold_ref[...], preferred_element_type=F32)


def _finalize_con(parts_num, parts_dc, skip, lp, wkv, bkv, w1, b1, w2, b2,
                  wfold):
    nb = NC // BR_N
    return pl.pallas_call(
        _finalize_con_body,
        out_shape=[jax.ShapeDtypeStruct((NC, DV), F32),
                   jax.ShapeDtypeStruct((NC, 2 * DV), F32),
                   jax.ShapeDtypeStruct((NC, DV), F32)],
        grid=(nb,),
        in_specs=[
            pl.BlockSpec((BR_N, DV), lambda i: (i, 0)),
            pl.BlockSpec((BR_N, DV), lambda i, nb=nb: (i + nb, 0)),
            pl.BlockSpec((BR_N, WDC), lambda i: (i, 0)),
            pl.BlockSpec((BR_N, WDC), lambda i, nb=nb: (i + nb, 0)),
            pl.BlockSpec((BR_N, DV), lambda i: (i, 0)),
            pl.BlockSpec((BR_N, NLP), lambda i: (i, 0)),
            pl.BlockSpec((DV, 2 * DV), lambda i: (0, 0)),
            pl.BlockSpec((NLP, 2 * DV), lambda i: (0, 0)),
            pl.BlockSpec((1, 2 * DV), lambda i: (0, 0)),
            pl.BlockSpec((DV, DV), lambda i: (0, 0)),
            pl.BlockSpec((NLP, DV), lambda i: (0, 0)),
            pl.BlockSpec((1, DV), lambda i: (0, 0)),
            pl.BlockSpec((DV, DV), lambda i: (0, 0)),
            pl.BlockSpec((1, DV), lambda i: (0, 0)),
            pl.BlockSpec((DV, DV), lambda i: (0, 0)),
        ],
        out_specs=[pl.BlockSpec((BR_N, DV), lambda i: (i, 0)),
                   pl.BlockSpec((BR_N, 2 * DV), lambda i: (i, 0)),
                   pl.BlockSpec((BR_N, DV), lambda i: (i, 0))],
    )(parts_num, parts_num, parts_dc, parts_dc, skip, lp,
      wkv[:DV], wkv[DV:], bkv.reshape(1, -1),
      w1[:DV], w1[DV:], b1.reshape(1, -1), w2, b2.reshape(1, -1), wfold)


def _finalize_var(parts_num, parts_dc, skip, lp, w1, b1, w2, b2, wfold):
    nbh = HALF // BR_N

    def pmap(i, nbh=nbh):
        return (jnp.where(i < nbh, i, i + 1), 0)

    return pl.pallas_call(
        _finalize_var_body,
        out_shape=[jax.ShapeDtypeStruct((NV, DV), F32),
                   jax.ShapeDtypeStruct((NV, DV), F32)],
        grid=(NV // BR_N,),
        in_specs=[
            pl.BlockSpec((BR_N, DV), pmap),
            pl.BlockSpec((BR_N, WDC), pmap),
            pl.BlockSpec((BR_N, DV), lambda i: (i, 0)),
            pl.BlockSpec((BR_N, NLP), lambda i: (i, 0)),
            pl.BlockSpec((DV, DV), lambda i: (0, 0)),
            pl.BlockSpec((NLP, DV), lambda i: (0, 0)),
            pl.BlockSpec((1, DV), lambda i: (0, 0)),
            pl.BlockSpec((DV, DV), lambda i: (0, 0)),
            pl.BlockSpec((1, DV), lambda i: (0, 0)),
            pl.BlockSpec((DV, DV), lambda i: (0, 0)),
        ],
        out_specs=[pl.BlockSpec((BR_N, DV), lambda i: (i, 0)),
                   pl.BlockSpec((BR_N, DV), lambda i: (i, 0))],
    )(parts_num, parts_dc, skip, lp,
      w1[:DV], w1[DV:], b1.reshape(1, -1), w2, b2.reshape(1, -1), wfold)


def _p3_body(el_ref, elp_ref, wa_ref, wb_ref, b_ref, oc_ref, ov_ref, og_ref):
    y = (jnp.dot(el_ref[...], wa_ref[...], preferred_element_type=F32)
         + jnp.dot(elp_ref[...], wb_ref[...], preferred_element_type=F32)
         + b_ref[...])
    oc_ref[...] = y[:, :128]
    ov_ref[...] = y[:, 128:256]
    og_ref[...] = y[:, 256:384]


def _p3_packed(el8, elp24, WA, WB, bias):
    n = E // 8
    return pl.pallas_call(
        _p3_body,
        out_shape=[jax.ShapeDtypeStruct((n, 128), F32)] * 3,
        grid=(n // BRP,),
        in_specs=[
            pl.BlockSpec((BRP, 128), lambda i: (i, 0)),
            pl.BlockSpec((BRP, 24), lambda i: (i, 0)),
            pl.BlockSpec((128, 384), lambda i: (0, 0)),
            pl.BlockSpec((24, 384), lambda i: (0, 0)),
            pl.BlockSpec((1, 384), lambda i: (0, 0)),
        ],
        out_specs=[pl.BlockSpec((BRP, 128), lambda i: (i, 0))] * 3,
    )(el8, elp24, WA, WB, bias)


def _k6_body(g1_ref, s2_ref, w_ref, b_ref, o_ref):
    h = jax.nn.relu(g1_ref[...] + s2_ref[...])
    o_ref[...] = jax.nn.relu(
        jnp.dot(h, w_ref[...], preferred_element_type=F32) + b_ref[...])


def _k6_packed(g1pk, s2pk, bdw2, b2t):
    n = E // 8
    return pl.pallas_call(
        _k6_body,
        out_shape=jax.ShapeDtypeStruct((n, 128), F32),
        grid=(n // BRP,),
        in_specs=[
            pl.BlockSpec((BRP, 128), lambda i: (i, 0)),
            pl.BlockSpec((BRP, 128), lambda i: (i, 0)),
            pl.BlockSpec((128, 128), lambda i: (0, 0)),
            pl.BlockSpec((1, 128), lambda i: (0, 0)),
        ],
        out_specs=pl.BlockSpec((BRP, 128), lambda i: (i, 0)),
    )(g1pk, s2pk, bdw2, b2t)


_SC_PARAMS = pltpu.CompilerParams(use_tc_tiling_on_sc=False,
                                  needs_layout_passes=False)


def _sc_mesh():
    info = plsc.get_sparse_core_info()
    return (plsc.VectorSubcoreMesh(core_axis_name="c", subcore_axis_name="s"),
            info.num_cores, info.num_subcores)


def _sc_attention(kv_tab, q_tab, epk, idx_kv, idx_q, zeros_n, zeros_d,
                  half_mode):
    mesh, ncores, nsub = _sc_mesh()
    nacc = VACC if half_mode else NC
    rpt = nacc // nsub
    ne_t = E // nsub if half_mode else E // (ncores * nsub)
    nch = ne_t // CHUNK

    def body(kv_hbm, q_hbm, ep_hbm, ikv_hbm, iq_hbm, zn_hbm, zd_hbm,
             on_hbm, od_hbm, ikv, iq, isc, kvb, qb, ecb, stg, dc2, exb,
             sem1, sem2, acc_n, acc_d):
        cid = lax.axis_index("c")
        sid = lax.axis_index("s")
        pltpu.sync_copy(zn_hbm.at[pl.ds(sid * rpt, rpt)],
                        acc_n.at[pl.ds(sid * rpt, rpt)])
        pltpu.sync_copy(zd_hbm.at[pl.ds(sid * rpt, rpt)],
                        acc_d.at[pl.ds(sid * rpt, rpt)])
        lanes = lax.iota(jnp.int32, 16)
        lane15 = lanes == 15
        ones16 = jnp.full((16,), 1.0, F32)
        zeros16 = jnp.zeros((16,), F32)

        def pre(g2, _):
            rows = g2 * 16 + lanes
            plsc.store_scatter(dc2, [rows, jnp.full((16,), 1, jnp.int32)],
                               ones16)
            for c in range(2, WDC):
                plsc.store_scatter(dc2, [rows, jnp.full((16,), c, jnp.int32)],
                                   zeros16)
            return 0

        lax.fori_loop(0, CHUNK // 16, pre, 0)
        plsc.subcore_barrier()

        if half_mode:
            tile_base = sid * ne_t
        else:
            tile_base = (sid * ncores + cid) * ne_t

        def step(g, _):
            base = tile_base + g * CHUNK
            pltpu.sync_copy(ikv_hbm.at[pl.ds(base, CHUNK)], ikv.at[0])
            pltpu.sync_copy(iq_hbm.at[pl.ds(base, CHUNK)], iq.at[0])
            c1 = pltpu.async_copy(kv_hbm.at[ikv.at[0]], kvb, sem1)
            c2 = pltpu.async_copy(q_hbm.at[iq.at[0]], qb, sem2)
            pltpu.sync_copy(ep_hbm.at[pl.ds(tile_base // 8 + g * (CHUNK // 8),
                                            CHUNK // 8)], ecb)
            c1.wait()
            c2.wait()

            def grp(g2, _):
                if half_mode:
                    iv = iq[0, pl.ds(g2 * 16, 16)]
                    lv = iv - cid * HALF
                    ok = (lv >= 0) & (lv < HALF)
                    isc[0, pl.ds(g2 * 16, 16)] = jnp.where(ok, lv, HALF)
                for j in range(16):
                    e = g2 * 16 + j
                    r6 = g2 * 2 + j // 8
                    co = (j % 8) * 16
                    ecv = ecb[r6, pl.ds(co, 16)]
                    kj = kvb[e, pl.ds(0, 16)] + ecv
                    pr = plsc.cumsum(qb[e] * kj * 0.25)
                    exv16 = jnp.exp(pr)
                    plsc.store_compressed(exb.at[pl.ds(e, 16)], exv16,
                                          mask=lane15)
                exv = exb[pl.ds(g2 * 16, 16)]
                plsc.store_scatter(dc2, [g2 * 16 + lanes,
                                         jnp.zeros((16,), jnp.int32)], exv)
                for j in range(16):
                    e = g2 * 16 + j
                    r6 = g2 * 2 + j // 8
                    co = (j % 8) * 16
                    ecv = ecb[r6, pl.ds(co, 16)]
                    vj = kvb[e, pl.ds(16, 16)] + ecv
                    exs = exb[pl.ds(e, 16)][0]
                    stg[e, :] = vj * exs
                return 0

            lax.fori_loop(0, CHUNK // 16, grp, 0)
            sc_idx = isc.at[0] if half_mode else iq.at[0]
            pltpu.sync_copy(stg, acc_n.at[sc_idx], add=True)
            pltpu.sync_copy(dc2, acc_d.at[sc_idx], add=True)
            return 0

        lax.fori_loop(0, nch, step, 0)
        plsc.subcore_barrier()
        pltpu.sync_copy(acc_n.at[pl.ds(sid * rpt, rpt)],
                        on_hbm.at[pl.ds(cid * nacc + sid * rpt, rpt)])
        pltpu.sync_copy(acc_d.at[pl.ds(sid * rpt, rpt)],
                        od_hbm.at[pl.ds(cid * nacc + sid * rpt, rpt)])

    return pl.kernel(
        body,
        out_type=[jax.ShapeDtypeStruct((2 * nacc, DV), F32),
                  jax.ShapeDtypeStruct((2 * nacc, WDC), F32)],
        mesh=mesh,
        compiler_params=_SC_PARAMS,
        scratch_types=[
            pltpu.VMEM((1, CHUNK), jnp.int32),
            pltpu.VMEM((1, CHUNK), jnp.int32),
            pltpu.VMEM((1, CHUNK), jnp.int32),
            pltpu.VMEM((CHUNK, 2 * DV), F32),
            pltpu.VMEM((CHUNK, DV), F32),
            pltpu.VMEM((CHUNK // 8, 128), F32),
            pltpu.VMEM((CHUNK, DV), F32),
            pltpu.VMEM((CHUNK, WDC), F32),
            pltpu.VMEM((CHUNK + 16,), F32),
            pltpu.SemaphoreType.DMA,
            pltpu.SemaphoreType.DMA,
            pltpu.VMEM_SHARED((nacc, DV), F32),
            pltpu.VMEM_SHARED((nacc, WDC), F32),
        ],
    )(kv_tab, q_tab, epk, idx_kv, idx_q, zeros_n, zeros_d)


def _sc_s2(vc2, cc2, src, dst):
    mesh, ncores, nsub = _sc_mesh()
    ne_t = E // (ncores * nsub)
    nch = ne_t // CHUNK

    def body(v_hbm, c_hbm, src_hbm, dst_hbm, o_hbm,
             isv, idv, vb, cb, stg2, sem1, sem2):
        cid = lax.axis_index("c")
        sid = lax.axis_index("s")
        tile_base = (sid * ncores + cid) * ne_t

        def step(g, _):
            base = tile_base + g * CHUNK
            pltpu.sync_copy(src_hbm.at[pl.ds(base, CHUNK)], isv.at[0])
            pltpu.sync_copy(dst_hbm.at[pl.ds(base, CHUNK)], idv.at[0])
            c1 = pltpu.async_copy(v_hbm.at[isv.at[0]], vb, sem1)
            c2 = pltpu.async_copy(c_hbm.at[idv.at[0]], cb, sem2)
            c1.wait()
            c2.wait()

            def grp(g2, _):
                for j in range(16):
                    e = g2 * 16 + j
                    r6 = g2 * 2 + j // 8
                    co = (j % 8) * 16
                    stg2[r6, pl.ds(co, 16)] = vb[e] + cb[e]
                return 0

            lax.fori_loop(0, CHUNK // 16, grp, 0)
            pltpu.sync_copy(stg2, o_hbm.at[pl.ds(tile_base // 8
                                                 + g * (CHUNK // 8),
                                                 CHUNK // 8)])
            return 0

        lax.fori_loop(0, nch, step, 0)

    return pl.kernel(
        body,
        out_type=jax.ShapeDtypeStruct((E // 8, 128), F32),
        mesh=mesh,
        compiler_params=_SC_PARAMS,
        scratch_types=[
            pltpu.VMEM((1, CHUNK), jnp.int32),
            pltpu.VMEM((1, CHUNK), jnp.int32),
            pltpu.VMEM((CHUNK, DV), F32),
            pltpu.VMEM((CHUNK, DV), F32),
            pltpu.VMEM((CHUNK // 8, 128), F32),
            pltpu.SemaphoreType.DMA,
            pltpu.SemaphoreType.DMA,
        ],
    )(vc2, cc2, src, dst)


def kernel(var_lp_f, con_lp_f, edge_lp_f_wo_ss, var_learned_f, con_learned_f,
           edge_learned_f, params, edge_index_var_con, batch_index_var,
           batch_index_con, batch_index_edge):
    p = params[0]
    src = edge_index_var_con[0]
    dst = edge_index_var_con[1]

    rnorm = _rnorm(var_lp_f[:, 2].reshape(800, 125))
    s = rnorm[0, 0]
    eye8 = jnp.eye(8, dtype=F32)

    def s18(W):
        return W.at[18].set(W[18] * s)

    def s2row(W):
        return W.at[2].set(W[2] * s)

    W64 = jnp.concatenate([s18(p["con"]["k"]["W"]), s18(p["con"]["v"]["W"]),
                           s18(p["var"]["q"]["W"]), s18(p["var"]["skip"]["W"])],
                          axis=1)
    b64 = jnp.concatenate([p["con"]["k"]["b"], p["con"]["v"]["b"],
                           p["var"]["q"]["b"], p["var"]["skip"]["b"]])
    W32 = jnp.concatenate([s18(p["con"]["q"]["W"]), s18(p["con"]["skip"]["W"])],
                          axis=1)
    b32 = jnp.concatenate([p["con"]["q"]["b"], p["con"]["skip"]["b"]])
    W1 = p["eu_e1"]["W"]
    W1a, W1b, W1c = W1[:19], W1[19:35], W1[35:51]
    Wkv_c = jnp.concatenate([s18(p["var"]["k"]["W"]), s18(p["var"]["v"]["W"])],
                            axis=1)
    bkv_c = jnp.concatenate([p["var"]["k"]["b"], p["var"]["v"]["b"]])

    Wea = jnp.concatenate([p["con"]["e"]["W"][:DV], p["var"]["e"]["W"][:DV],
                           W1a[:DV]], axis=1)
    Web = jnp.concatenate([s2row(p["con"]["e"]["W"][DV:]),
                           s2row(p["var"]["e"]["W"][DV:]),
                           s2row(W1a[DV:])], axis=1)
    BDA = jnp.concatenate([jnp.kron(eye8, Wea[:, :16]),
                           jnp.kron(eye8, Wea[:, 16:32]),
                           jnp.kron(eye8, Wea[:, 32:])], axis=1)
    BDB = jnp.concatenate([jnp.kron(eye8, Web[:, :16]),
                           jnp.kron(eye8, Web[:, 16:32]),
                           jnp.kron(eye8, Web[:, 32:])], axis=1)
    b384 = jnp.concatenate([jnp.zeros((256,), F32),
                            jnp.tile(p["eu_e1"]["b"], 8)]).reshape(1, 384)
    BDW2 = jnp.kron(eye8, p["eu_e2"]["W"])
    b2t = jnp.tile(p["eu_e2"]["b"], 8).reshape(1, 128)

    kv_v, q_v, skip_v = _proj(var_learned_f, var_lp_f, W64, b64,
                              (2 * DV, DV, DV), BR_N)
    q_c, skip_c = _proj(con_learned_f, con_lp_f, W32, b32, (DV, DV), BR_N)

    el8 = edge_learned_f.reshape(E // 8, 128)
    elp24 = edge_lp_f_wo_ss.reshape(E // 8, 24)
    ec_pk, ev_pk, g1_pk = _p3_packed(el8, elp24, BDA, BDB, b384)

    zc_n = jnp.zeros((NC, DV), F32)
    zc_d = jnp.zeros((NC, WDC), F32)
    zv_n = jnp.zeros((VACC, DV), F32)
    zv_d = jnp.zeros((VACC, WDC), F32)

    pc_num, pc_dc = _sc_attention(kv_v, q_c, ec_pk, src, dst, zc_n, zc_d,
                                  half_mode=False)
    con_l, kv_c, cc2 = _finalize_con(pc_num, pc_dc, skip_c, con_lp_f,
                                     Wkv_c, bkv_c,
                                     s18(p["eu_c1"]["W"]), p["eu_c1"]["b"],
                                     p["eu_c2"]["W"], p["eu_c2"]["b"], W1c)

    pv_num, pv_dc = _sc_attention(kv_c, q_v, ev_pk, dst, src, zv_n, zv_d,
                                  half_mode=True)
    var_l, vc2 = _finalize_var(pv_num, pv_dc, skip_v, var_lp_f,
                               s18(p["eu_v1"]["W"]), p["eu_v1"]["b"],
                               p["eu_v2"]["W"], p["eu_v2"]["b"], W1b)

    s2pk = _sc_s2(vc2, cc2, src, dst)
    epk = _k6_packed(g1_pk, s2pk, BDW2, b2t)
    edge_l = epk.reshape(E, DV)

    return (var_l, con_l, edge_l)


# speedup vs baseline: 12.1324x; 1.8006x over previous
import functools

import jax
import jax.numpy as jnp
from jax import lax
from jax.experimental import pallas as pl
from jax.experimental.pallas import tpu as pltpu
from jax.experimental.pallas import tpu_sc as plsc

NV = 100000
NC = 50000
E = 1600000
DV = 16
NLP = 3

WDC = 8
CHUNK = 400
W32, L32 = 50400, 37600
W16, L16 = 100800, 88000
HALF = NV // 2
VACC = HALF + 2000
BR_N = 2000
BRE = 16384
NBE = -(-E // BRE)
EPAD = NBE * BRE

F32 = jnp.float32


def _norm_body(x_ref, o_ref):
    x = x_ref[...]
    r = 1.0 / jnp.maximum(jnp.sqrt(jnp.sum(x * x)), 1e-6)
    o_ref[...] = jnp.broadcast_to(r, (1, 1))


def _rnorm(col):
    return pl.pallas_call(
        _norm_body,
        out_shape=jax.ShapeDtypeStruct((1, 1), F32),
        in_specs=[pl.BlockSpec((800, 125), lambda: (0, 0))],
        out_specs=pl.BlockSpec((1, 1), lambda: (0, 0)),
    )(col)


def _proj_body(splits, xl_ref, xp_ref, wa_ref, wb_ref, b_ref, *o_refs):
    y = (jnp.dot(xl_ref[...], wa_ref[...], preferred_element_type=F32)
         + jnp.dot(xp_ref[...], wb_ref[...], preferred_element_type=F32)
         + b_ref[...])
    off = 0
    for o_ref, w in zip(o_refs, splits):
        o_ref[...] = y[:, off:off + w]
        off += w


def _proj(xl, xp, W, b, splits, br):
    n, kl = xl.shape
    kp = xp.shape[1]
    f = W.shape[1]
    wa, wb = W[:kl], W[kl:]
    return pl.pallas_call(
        functools.partial(_proj_body, splits),
        out_shape=[jax.ShapeDtypeStruct((n, w), F32) for w in splits],
        grid=(n // br,),
        in_specs=[
            pl.BlockSpec((br, kl), lambda i: (i, 0)),
            pl.BlockSpec((br, kp), lambda i: (i, 0)),
            pl.BlockSpec((kl, f), lambda i: (0, 0)),
            pl.BlockSpec((kp, f), lambda i: (0, 0)),
            pl.BlockSpec((1, f), lambda i: (0, 0)),
        ],
        out_specs=[pl.BlockSpec((br, w), lambda i: (i, 0)) for w in splits],
    )(xl, xp, wa, wb, b.reshape(1, f))


def _finalize_con_body(p0_ref, p1_ref, d0_ref, d1_ref, skip_ref, lp_ref,
                       wkva_ref, wkvb_ref, bkv_ref, w1a_ref, w1b_ref, b1_ref,
                       w2_ref, b2_ref, wfold_ref, o_node_ref, o_kv_ref,
                       o_fold_ref):
    num = p0_ref[...] + p1_ref[...]
    dc = d0_ref[...] + d1_ref[...]
    den = dc[:, 0:1]
    cnt = dc[:, 1:2]
    node = jax.nn.relu(num / (den + 1e-16) / jnp.maximum(cnt, 1.0)
                       + skip_ref[...])
    o_node_ref[...] = node
    lp = lp_ref[...]
    o_kv_ref[...] = (jnp.dot(node, wkva_ref[...], preferred_element_type=F32)
                     + jnp.dot(lp, wkvb_ref[...], preferred_element_type=F32)
                     + bkv_ref[...])
    h = jax.nn.relu(jnp.dot(node, w1a_ref[...], preferred_element_type=F32)
                    + jnp.dot(lp, w1b_ref[...], preferred_element_type=F32)
                    + b1_ref[...])
    h2 = jax.nn.relu(jnp.dot(h, w2_ref[...], preferred_element_type=F32)
                     + b2_ref[...])
    o_fold_ref[...] = jnp.dot(h2, wfold_ref[...], preferred_element_type=F32)


def _finalize_var_body(p0_ref, d0_ref, skip_ref, lp_ref,
                       w1a_ref, w1b_ref, b1_ref, w2_ref, b2_ref, wfold_ref,
                       o_node_ref, o_fold_ref):
    num = p0_ref[...]
    dc = d0_ref[...]
    den = dc[:, 0:1]
    cnt = dc[:, 1:2]
    node = jax.nn.relu(num / (den + 1e-16) / jnp.maximum(cnt, 1.0)
                       + skip_ref[...])
    o_node_ref[...] = node
    lp = lp_ref[...]
    h = jax.nn.relu(jnp.dot(node, w1a_ref[...], preferred_element_type=F32)
                    + jnp.dot(lp, w1b_ref[...], preferred_element_type=F32)
                    + b1_ref[...])
    h2 = jax.nn.relu(jnp.dot(h, w2_ref[...], preferred_element_type=F32)
                     + b2_ref[...])
    o_fold_ref[...] = jnp.dot(h2, wfold_ref[...], preferred_element_type=F32)


def _finalize_con(parts_num, parts_dc, skip, lp, wkv, bkv, w1, b1, w2, b2,
                  wfold):
    nb = NC // BR_N
    return pl.pallas_call(
        _finalize_con_body,
        out_shape=[jax.ShapeDtypeStruct((NC, DV), F32),
                   jax.ShapeDtypeStruct((NC, 2 * DV), F32),
                   jax.ShapeDtypeStruct((NC, DV), F32)],
        grid=(nb,),
        in_specs=[
            pl.BlockSpec((BR_N, DV), lambda i: (i, 0)),
            pl.BlockSpec((BR_N, DV), lambda i, nb=nb: (i + nb, 0)),
            pl.BlockSpec((BR_N, WDC), lambda i: (i, 0)),
            pl.BlockSpec((BR_N, WDC), lambda i, nb=nb: (i + nb, 0)),
            pl.BlockSpec((BR_N, DV), lambda i: (i, 0)),
            pl.BlockSpec((BR_N, NLP), lambda i: (i, 0)),
            pl.BlockSpec((DV, 2 * DV), lambda i: (0, 0)),
            pl.BlockSpec((NLP, 2 * DV), lambda i: (0, 0)),
            pl.BlockSpec((1, 2 * DV), lambda i: (0, 0)),
            pl.BlockSpec((DV, DV), lambda i: (0, 0)),
            pl.BlockSpec((NLP, DV), lambda i: (0, 0)),
            pl.BlockSpec((1, DV), lambda i: (0, 0)),
            pl.BlockSpec((DV, DV), lambda i: (0, 0)),
            pl.BlockSpec((1, DV), lambda i: (0, 0)),
            pl.BlockSpec((DV, DV), lambda i: (0, 0)),
        ],
        out_specs=[pl.BlockSpec((BR_N, DV), lambda i: (i, 0)),
                   pl.BlockSpec((BR_N, 2 * DV), lambda i: (i, 0)),
                   pl.BlockSpec((BR_N, DV), lambda i: (i, 0))],
    )(parts_num, parts_num, parts_dc, parts_dc, skip, lp,
      wkv[:DV], wkv[DV:], bkv.reshape(1, -1),
      w1[:DV], w1[DV:], b1.reshape(1, -1), w2, b2.reshape(1, -1), wfold)


def _finalize_var(parts_num, parts_dc, skip, lp, w1, b1, w2, b2, wfold):
    nbh = HALF // BR_N

    def pmap(i, nbh=nbh):
        return (jnp.where(i < nbh, i, i + 1), 0)

    return pl.pallas_call(
        _finalize_var_body,
        out_shape=[jax.ShapeDtypeStruct((NV, DV), F32),
                   jax.ShapeDtypeStruct((NV, DV), F32)],
        grid=(NV // BR_N,),
        in_specs=[
            pl.BlockSpec((BR_N, DV), pmap),
            pl.BlockSpec((BR_N, WDC), pmap),
            pl.BlockSpec((BR_N, DV), lambda i: (i, 0)),
            pl.BlockSpec((BR_N, NLP), lambda i: (i, 0)),
            pl.BlockSpec((DV, DV), lambda i: (0, 0)),
            pl.BlockSpec((NLP, DV), lambda i: (0, 0)),
            pl.BlockSpec((1, DV), lambda i: (0, 0)),
            pl.BlockSpec((DV, DV), lambda i: (0, 0)),
            pl.BlockSpec((1, DV), lambda i: (0, 0)),
            pl.BlockSpec((DV, DV), lambda i: (0, 0)),
        ],
        out_specs=[pl.BlockSpec((BR_N, DV), lambda i: (i, 0)),
                   pl.BlockSpec((BR_N, DV), lambda i: (i, 0))],
    )(parts_num, parts_dc, skip, lp,
      w1[:DV], w1[DV:], b1.reshape(1, -1), w2, b2.reshape(1, -1), wfold)


def _p3t_body(el_ref, elp_ref, wta_ref, wtb_ref, b_ref, *o_refs):
    y = (jnp.dot(wta_ref[...], el_ref[...], preferred_element_type=F32)
         + jnp.dot(wtb_ref[...], elp_ref[...], preferred_element_type=F32)
         + b_ref[...])
    for f in range(2 * DV):
        o_refs[f][...] = y[f, :]
    o_refs[2 * DV][...] = y[2 * DV:, :]


def _p3t(elT, elpT, WT, bias):
    return pl.pallas_call(
        _p3t_body,
        out_shape=[jax.ShapeDtypeStruct((EPAD,), F32)] * (2 * DV)
        + [jax.ShapeDtypeStruct((DV, E), F32)],
        grid=(NBE,),
        in_specs=[
            pl.BlockSpec((DV, BRE), lambda i: (0, i)),
            pl.BlockSpec((NLP, BRE), lambda i: (0, i)),
            pl.BlockSpec((48, DV), lambda i: (0, 0)),
            pl.BlockSpec((48, NLP), lambda i: (0, 0)),
            pl.BlockSpec((48, 1), lambda i: (0, 0)),
        ],
        out_specs=[pl.BlockSpec((BRE,), lambda i: (i,))] * (2 * DV)
        + [pl.BlockSpec((DV, BRE), lambda i: (0, i))],
    )(elT, elpT, WT[:, :DV], WT[:, DV:], bias.reshape(48, 1))


def _k6t_body(g1_ref, *refs):
    s2_refs = refs[:DV]
    w_ref, b_ref, o_ref = refs[DV], refs[DV + 1], refs[DV + 2]
    s2 = jnp.stack([r[...] for r in s2_refs], axis=0)
    h = jax.nn.relu(g1_ref[...] + s2)
    o_ref[...] = jax.nn.relu(
        jnp.dot(w_ref[...], h, preferred_element_type=F32) + b_ref[...])


def _k6t(g1T, s2s, w2t, b2col):
    return pl.pallas_call(
        _k6t_body,
        out_shape=jax.ShapeDtypeStruct((DV, E), F32),
        grid=(NBE,),
        in_specs=[pl.BlockSpec((DV, BRE), lambda i: (0, i))]
        + [pl.BlockSpec((BRE,), lambda i: (i,))] * DV
        + [
            pl.BlockSpec((DV, DV), lambda i: (0, 0)),
            pl.BlockSpec((DV, 1), lambda i: (0, 0)),
        ],
        out_specs=pl.BlockSpec((DV, BRE), lambda i: (0, i)),
    )(g1T, *s2s, w2t, b2col)


_SC_PARAMS = pltpu.CompilerParams(use_tc_tiling_on_sc=False,
                                  needs_layout_passes=False)


def _sc_mesh():
    info = plsc.get_sparse_core_info()
    return (plsc.VectorSubcoreMesh(core_axis_name="c", subcore_axis_name="s"),
            info.num_cores, info.num_subcores)


def _sc_attention(kv_tab, q_tab, epT, idx_kv, idx_q, zeros_n, zeros_d,
                  half_mode):
    mesh, ncores, nsub = _sc_mesh()
    nacc = VACC if half_mode else NC
    rpt = nacc // nsub

    def body(kv_hbm, q_hbm, *refs):
        ep_hbms = refs[:DV]
        (ikv_hbm, iq_hbm, zn_hbm, zd_hbm, on_hbm, od_hbm,
         ikv, iq, isc, kvb, qb, ecb, stg, dc2,
         sem1, sem2, sem3, acc_n, acc_d) = refs[DV:]
        cid = lax.axis_index("c")
        sid = lax.axis_index("s")
        pltpu.sync_copy(zn_hbm.at[pl.ds(sid * rpt, rpt)],
                        acc_n.at[pl.ds(sid * rpt, rpt)])
        pltpu.sync_copy(zd_hbm.at[pl.ds(sid * rpt, rpt)],
                        acc_d.at[pl.ds(sid * rpt, rpt)])
        lanes = lax.iota(jnp.int32, 16)
        ones16 = jnp.full((16,), 1.0, F32)
        zeros16 = jnp.zeros((16,), F32)
        zcol = jnp.zeros((16,), jnp.int32)

        def pre(g2, _):
            rows = g2 * 16 + lanes
            plsc.store_scatter(dc2, [rows, jnp.full((16,), 1, jnp.int32)],
                               ones16)
            for c in range(2, WDC):
                plsc.store_scatter(dc2, [rows, jnp.full((16,), c, jnp.int32)],
                                   zeros16)
            return 0

        lax.fori_loop(0, CHUNK // 16, pre, 0)
        plsc.subcore_barrier()

        if half_mode:
            tile_base = sid * W16
            nch = jnp.where(sid == nsub - 1, L16 // CHUNK, W16 // CHUNK)
        else:
            wid = sid * ncores + cid
            tile_base = wid * W32
            nch = jnp.where(wid == ncores * nsub - 1, L32 // CHUNK,
                            W32 // CHUNK)

        def step(g, _):
            base = tile_base + g * CHUNK
            pltpu.sync_copy(ikv_hbm.at[pl.ds(base, CHUNK)], ikv.at[0])
            pltpu.sync_copy(iq_hbm.at[pl.ds(base, CHUNK)], iq.at[0])
            c1 = pltpu.async_copy(kv_hbm.at[ikv.at[0]], kvb, sem1)
            c2 = pltpu.async_copy(q_hbm.at[iq.at[0]], qb, sem2)
            cps = [pltpu.async_copy(ep_hbms[f].at[pl.ds(base, CHUNK)],
                                    ecb.at[f], sem3) for f in range(DV)]
            c1.wait()
            c2.wait()
            for c in cps:
                c.wait()

            def grp(g2, _):
                e0 = g2 * 16
                rows = e0 + lanes
                if half_mode:
                    iv = iq[0, pl.ds(e0, 16)]
                    lv = iv - cid * HALF
                    ok = (lv >= 0) & (lv < HALF)
                    isc[0, pl.ds(e0, 16)] = jnp.where(ok, lv, HALF)
                acc = [zeros16, zeros16, zeros16, zeros16]
                for f in range(DV):
                    qf = plsc.load_gather(qb, [rows, jnp.full((16,), f, jnp.int32)])
                    kf = plsc.load_gather(kvb, [rows, jnp.full((16,), f, jnp.int32)])
                    ef = ecb[f, pl.ds(e0, 16)]
                    acc[f % 4] = acc[f % 4] + qf * (kf + ef)
                ex16 = jnp.exp(((acc[0] + acc[1]) + (acc[2] + acc[3])) * 0.25)
                plsc.store_scatter(dc2, [rows, zcol], ex16)
                for f in range(DV):
                    vf = plsc.load_gather(kvb, [rows, jnp.full((16,), DV + f, jnp.int32)])
                    ef = ecb[f, pl.ds(e0, 16)]
                    plsc.store_scatter(stg, [rows, jnp.full((16,), f, jnp.int32)],
                                       ex16 * (vf + ef))
                return 0

            lax.fori_loop(0, CHUNK // 16, grp, 0)
            sc_idx = isc.at[0] if half_mode else iq.at[0]
            pltpu.sync_copy(stg, acc_n.at[sc_idx], add=True)
            pltpu.sync_copy(dc2, acc_d.at[sc_idx], add=True)
            return 0

        lax.fori_loop(0, nch, step, 0)
        plsc.subcore_barrier()
        pltpu.sync_copy(acc_n.at[pl.ds(sid * rpt, rpt)],
                        on_hbm.at[pl.ds(cid * nacc + sid * rpt, rpt)])
        pltpu.sync_copy(acc_d.at[pl.ds(sid * rpt, rpt)],
                        od_hbm.at[pl.ds(cid * nacc + sid * rpt, rpt)])

    return pl.kernel(
        body,
        out_type=[jax.ShapeDtypeStruct((2 * nacc, DV), F32),
                  jax.ShapeDtypeStruct((2 * nacc, WDC), F32)],
        mesh=mesh,
        compiler_params=_SC_PARAMS,
        scratch_types=[
            pltpu.VMEM((1, CHUNK), jnp.int32),
            pltpu.VMEM((1, CHUNK), jnp.int32),
            pltpu.VMEM((1, CHUNK), jnp.int32),
            pltpu.VMEM((CHUNK, 2 * DV), F32),
            pltpu.VMEM((CHUNK, DV), F32),
            pltpu.VMEM((DV, CHUNK), F32),
            pltpu.VMEM((CHUNK, DV), F32),
            pltpu.VMEM((CHUNK, WDC), F32),
            pltpu.SemaphoreType.DMA,
            pltpu.SemaphoreType.DMA,
            pltpu.SemaphoreType.DMA,
            pltpu.VMEM_SHARED((nacc, DV), F32),
            pltpu.VMEM_SHARED((nacc, WDC), F32),
        ],
    )(kv_tab, q_tab, *epT, idx_kv, idx_q, zeros_n, zeros_d)


def _sc_s2(vc2, cc2, src, dst):
    mesh, ncores, nsub = _sc_mesh()

    def body(v_hbm, c_hbm, src_hbm, dst_hbm, *refs):
        o_hbms = refs[:DV]
        isv, idv, vb, cb, stg2, sem1, sem2, sem3 = refs[DV:]
        cid = lax.axis_index("c")
        sid = lax.axis_index("s")
        wid = sid * ncores + cid
        tile_base = wid * W32
        nch = jnp.where(wid == ncores * nsub - 1, L32 // CHUNK, W32 // CHUNK)
        lanes = lax.iota(jnp.int32, 16)

        def step(g, _):
            base = tile_base + g * CHUNK
            pltpu.sync_copy(src_hbm.at[pl.ds(base, CHUNK)], isv.at[0])
            pltpu.sync_copy(dst_hbm.at[pl.ds(base, CHUNK)], idv.at[0])
            c1 = pltpu.async_copy(v_hbm.at[isv.at[0]], vb, sem1)
            c2 = pltpu.async_copy(c_hbm.at[idv.at[0]], cb, sem2)
            c1.wait()
            c2.wait()

            def grp(g2, _):
                e0 = g2 * 16
                rows = e0 + lanes
                for f in range(DV):
                    colf = jnp.full((16,), f, jnp.int32)
                    vf = plsc.load_gather(vb, [rows, colf])
                    cf = plsc.load_gather(cb, [rows, colf])
                    stg2[f, pl.ds(e0, 16)] = vf + cf
                return 0

            lax.fori_loop(0, CHUNK // 16, grp, 0)
            cps = [pltpu.async_copy(stg2.at[f], o_hbms[f].at[pl.ds(base, CHUNK)],
                                    sem3) for f in range(DV)]
            for c in cps:
                c.wait()
            return 0

        lax.fori_loop(0, nch, step, 0)

    return pl.kernel(
        body,
        out_type=[jax.ShapeDtypeStruct((EPAD,), F32)] * DV,
        mesh=mesh,
        compiler_params=_SC_PARAMS,
        scratch_types=[
            pltpu.VMEM((1, CHUNK), jnp.int32),
            pltpu.VMEM((1, CHUNK), jnp.int32),
            pltpu.VMEM((CHUNK, DV), F32),
            pltpu.VMEM((CHUNK, DV), F32),
            pltpu.VMEM((DV, CHUNK), F32),
            pltpu.SemaphoreType.DMA,
            pltpu.SemaphoreType.DMA,
            pltpu.SemaphoreType.DMA,
        ],
    )(vc2, cc2, src, dst)


def kernel(var_lp_f, con_lp_f, edge_lp_f_wo_ss, var_learned_f, con_learned_f,
           edge_learned_f, params, edge_index_var_con, batch_index_var,
           batch_index_con, batch_index_edge):
    p = params[0]
    src = edge_index_var_con[0]
    dst = edge_index_var_con[1]

    rnorm = _rnorm(var_lp_f[:, 2].reshape(800, 125))
    s = rnorm[0, 0]

    def s18(W):
        return W.at[18].set(W[18] * s)

    W64 = jnp.concatenate([s18(p["con"]["k"]["W"]), s18(p["con"]["v"]["W"]),
                           s18(p["var"]["q"]["W"]), s18(p["var"]["skip"]["W"])],
                          axis=1)
    b64 = jnp.concatenate([p["con"]["k"]["b"], p["con"]["v"]["b"],
                           p["var"]["q"]["b"], p["var"]["skip"]["b"]])
    W32 = jnp.concatenate([s18(p["con"]["q"]["W"]), s18(p["con"]["skip"]["W"])],
                          axis=1)
    b32 = jnp.concatenate([p["con"]["q"]["b"], p["con"]["skip"]["b"]])
    W1 = p["eu_e1"]["W"]
    W1a, W1b, W1c = W1[:19], W1[19:35], W1[35:51]
    Wkv_c = jnp.concatenate([s18(p["var"]["k"]["W"]), s18(p["var"]["v"]["W"])],
                            axis=1)
    bkv_c = jnp.concatenate([p["var"]["k"]["b"], p["var"]["v"]["b"]])

    W48 = jnp.concatenate([s18(p["con"]["e"]["W"]), s18(p["var"]["e"]["W"]),
                           s18(W1a)], axis=1)
    b48 = jnp.concatenate([jnp.zeros((32,), F32), p["eu_e1"]["b"]])

    kv_v, q_v, skip_v = _proj(var_learned_f, var_lp_f, W64, b64,
                              (2 * DV, DV, DV), BR_N)
    q_c, skip_c = _proj(con_learned_f, con_lp_f, W32, b32, (DV, DV), BR_N)

    elT = edge_learned_f.T
    elpT = edge_lp_f_wo_ss.T
    p3_out = _p3t(elT, elpT, W48.T, b48)
    ecT = p3_out[:DV]
    evT = p3_out[DV:2 * DV]
    g1T = p3_out[2 * DV]

    zc_n = jnp.zeros((NC, DV), F32)
    zc_d = jnp.zeros((NC, WDC), F32)
    zv_n = jnp.zeros((VACC, DV), F32)
    zv_d = jnp.zeros((VACC, WDC), F32)

    pc_num, pc_dc = _sc_attention(kv_v, q_c, ecT, src, dst, zc_n, zc_d,
                                  half_mode=False)
    con_l, kv_c, cc2 = _finalize_con(pc_num, pc_dc, skip_c, con_lp_f,
                                     Wkv_c, bkv_c,
                                     s18(p["eu_c1"]["W"]), p["eu_c1"]["b"],
                                     p["eu_c2"]["W"], p["eu_c2"]["b"], W1c)

    pv_num, pv_dc = _sc_attention(kv_c, q_v, evT, dst, src, zv_n, zv_d,
                                  half_mode=True)
    var_l, vc2 = _finalize_var(pv_num, pv_dc, skip_v, var_lp_f,
                               s18(p["eu_v1"]["W"]), p["eu_v1"]["b"],
                               p["eu_v2"]["W"], p["eu_v2"]["b"], W1b)

    s2s = _sc_s2(vc2, cc2, src, dst)
    eT = _k6t(g1T, s2s, p["eu_e2"]["W"].T, p["eu_e2"]["b"].reshape(DV, 1))
    edge_l = eT.T

    return (var_l, con_l, edge_l)
